# baseline (speedup 1.0000x reference)
import functools
import math

import jax
import jax.numpy as jnp
import numpy as np
from jax import lax
from jax.experimental import pallas as pl
from jax.experimental.pallas import tpu as pltpu
from jax.experimental.pallas import tpu_sc as plsc

_ANCH = np.array(
    [[[16.2, 14.4], [41.1, 33.3], [74.1, 57.6]],
     [[110.4, 84.0], [146.4, 107.1], [180.3, 132.6]],
     [[226.0, 129.3], [214.8, 188.8], [278.2, 173.3]]], dtype=np.float32)
_BAL = (4.0, 1.0, 0.25)
_GAMMA = 1.5
_ANCHOR_T = 4.0
_IMG = 640.0

_B = 32
_NA = 3
_NO = 5
_NT = 512
_HW = ((80, 80), (40, 40), (20, 20))
_SA = [ _ANCH[i] / (_IMG / max(h, w)) for i, (h, w) in enumerate(_HW) ]

_NROW = _NO * _NA
_NCAND = _NROW * _NT
_NW = 32
_GPW = _NO * _NCAND // _NW
_CHUNK = 120
_NCHUNK = _GPW // _CHUNK


def _prep_compute(t6_ref, g0, g1, g2, m0, m1, m2):
    img = t6_ref[0:1, :]
    x = t6_ref[2:3, :]
    y = t6_ref[3:4, :]
    w = t6_ref[4:5, :]
    h = t6_ref[5:6, :]
    r_i = lax.broadcasted_iota(jnp.int32, (_NROW, _NT), 0)
    k = r_i // _NA
    a = r_i % _NA
    b = img.astype(jnp.int32)

    for (gref, mref, (H, W), sa) in ((g0, m0, _HW[0], _SA[0]),
                                     (g1, m1, _HW[1], _SA[1]),
                                     (g2, m2, _HW[2], _SA[2])):
        Wf, Hf = jnp.float32(W), jnp.float32(H)
        gx = x * Wf
        gy = y * Hf
        gw = w * Wf
        gh = h * Hf
        aw = jnp.where(a == 0, sa[0, 0], jnp.where(a == 1, sa[1, 0], sa[2, 0]))
        ah = jnp.where(a == 0, sa[0, 1], jnp.where(a == 1, sa[1, 1], sa[2, 1]))
        rw = gw / aw
        rh = gh / ah
        rr = jnp.maximum(jnp.maximum(rw, 1.0 / rw), jnp.maximum(rh, 1.0 / rh))
        jsel = rr < _ANCHOR_T
        jj = ((gx - jnp.floor(gx)) < 0.5) & (gx > 1.0)
        kk = ((gy - jnp.floor(gy)) < 0.5) & (gy > 1.0)
        gxi = Wf - gx
        gyi = Hf - gy
        ll = ((gxi - jnp.floor(gxi)) < 0.5) & (gxi > 1.0)
        mm = ((gyi - jnp.floor(gyi)) < 0.5) & (gyi > 1.0)
        jm = ((k == 0) | ((k == 1) & jj) | ((k == 2) & kk)
              | ((k == 3) & ll) | ((k == 4) & mm))
        mask = (jm & jsel).astype(jnp.float32)
        offx = jnp.where(k == 1, 0.5, 0.0) + jnp.where(k == 3, -0.5, 0.0)
        offy = jnp.where(k == 2, 0.5, 0.0) + jnp.where(k == 4, -0.5, 0.0)
        gsx = gx - offx
        gsy = gy - offy
        gijx = gsx.astype(jnp.int32)
        gijy = gsy.astype(jnp.int32)
        gi = jnp.clip(gijx, 0, W - 1)
        gj = jnp.clip(gijy, 0, H - 1)
        tx = gx - gijx.astype(jnp.float32)
        ty = gy - gijy.astype(jnp.float32)
        base = ((b * 15 + a * _NO) * H + gj) * W + gi
        ch = lax.broadcasted_iota(jnp.int32, (_NO, _NROW, _NT), 0)
        gref[...] = base[None] + ch * (H * W)
        zero = jnp.zeros((_NROW, _NT), jnp.float32)
        mref[...] = jnp.stack(
            [mask, tx, ty,
             jnp.broadcast_to(gw, (_NROW, _NT)),
             jnp.broadcast_to(gh, (_NROW, _NT)),
             aw, ah, zero], axis=0)


def _prep(t6):
    shp_g = jax.ShapeDtypeStruct((_NO, _NROW, _NT), jnp.int32)
    shp_m = jax.ShapeDtypeStruct((8, _NROW, _NT), jnp.float32)
    return pl.pallas_call(
        _prep_compute,
        out_shape=(shp_g, shp_g, shp_g, shp_m, shp_m, shp_m),
    )(t6)


_OBJ_SHAPE = {6400: (50, 128), 1600: (8, 200), 400: (8, 50)}


def _objsum_body(p0r, p1r, p2r, s_ref):
    ci = pl.program_id(0)

    @pl.when(ci == 0)
    def _():
        s_ref[0, 0] = 0.0
        s_ref[0, 1] = 0.0
        s_ref[0, 2] = 0.0
        s_ref[0, 3] = 0.0

    s_ref[0, 0] += jnp.sum(_softplus(p0r[...]))
    s_ref[0, 1] += jnp.sum(_softplus(p1r[...]))
    s_ref[0, 2] += jnp.sum(_softplus(p2r[...]))


def _objsum(p0, p1, p2):
    r4, in_specs = [], []
    for p, (hh, ww) in zip((p0, p1, p2), _HW):
        s, l = _OBJ_SHAPE[hh * ww]
        r4.append(p.reshape(_B, 15, s, l))
        in_specs.append(
            pl.BlockSpec((_B, 1, s, l), lambda c: (0, 5 * c + 4, 0, 0)))
    return pl.pallas_call(
        _objsum_body,
        grid=(3,),
        in_specs=in_specs,
        out_specs=pl.BlockSpec(memory_space=pltpu.SMEM),
        out_shape=jax.ShapeDtypeStruct((1, 4), jnp.float32),
    )(*r4)


def _sc_gather_body(p0h, p1h, p2h, g0h, g1h, g2h, o0h, o1h, o2h,
                    i0, i1, i2, v0, v1, v2, sem):
    wid = lax.axis_index("s") * 2 + lax.axis_index("c")
    trip = ((p0h, g0h, o0h, i0, v0), (p1h, g1h, o1h, i1, v1),
            (p2h, g2h, o2h, i2, v2))
    lds = [pltpu.async_copy(gh.at[wid], iv, sem)
           for (_, gh, _, iv, _) in trip]
    for c in lds:
        c.wait()
    gs = [pltpu.async_copy(ph.at[iv.at[j]], vv.at[j], sem)
          for (ph, _, _, iv, vv) in trip
          for j in range(_NCHUNK)]
    for c in gs:
        c.wait()
    ws = [pltpu.async_copy(vv, oh.at[wid], sem)
          for (_, _, oh, _, vv) in trip]
    for c in ws:
        c.wait()


def _sc_gather(p0f, p1f, p2f, g0, g1, g2):
    mesh = plsc.VectorSubcoreMesh(core_axis_name="c", subcore_axis_name="s")
    out = jax.ShapeDtypeStruct((_NW, _NCHUNK, _CHUNK), jnp.float32)
    fn = functools.partial(
        pl.kernel,
        out_type=[out, out, out],
        mesh=mesh,
        scratch_types=[
            pltpu.VMEM((_NCHUNK, _CHUNK), jnp.int32),
            pltpu.VMEM((_NCHUNK, _CHUNK), jnp.int32),
            pltpu.VMEM((_NCHUNK, _CHUNK), jnp.int32),
            pltpu.VMEM((_NCHUNK, _CHUNK), jnp.float32),
            pltpu.VMEM((_NCHUNK, _CHUNK), jnp.float32),
            pltpu.VMEM((_NCHUNK, _CHUNK), jnp.float32),
            pltpu.SemaphoreType.DMA,
        ],
    )(_sc_gather_body)
    return fn(p0f, p1f, p2f, g0, g1, g2)


def _sigmoid(v):
    return 1.0 / (1.0 + jnp.exp(-v))


_ATAN_C = (1.0, -0.3333312, 0.19993716, -0.14213197, 0.10681421,
           -0.0759681, 0.0438556, -0.01682745, 0.003049968)


def _atan_pos(x):
    inv = x > 1.0
    z = jnp.where(inv, 1.0 / x, x)
    z2 = z * z
    acc = jnp.full_like(z2, _ATAN_C[-1])
    for c in _ATAN_C[-2::-1]:
        acc = acc * z2 + c
    r = z * acc
    return jnp.where(inv, (math.pi / 2) - r, r)


def _math_body(ps0, ps1, ps2, m0, m1, m2, out_ref):
    eps = jnp.float32(1e-7)
    for l, (psr, mr) in enumerate(((ps0, m0), (ps1, m1), (ps2, m2))):
        px, py, pw, ph, xo = (psr[c] for c in range(_NO))
        mask, tx, ty, tw, th, aw, ah = (mr[c] for c in range(7))
        bx = _sigmoid(px) * 2.0 - 0.5
        by = _sigmoid(py) * 2.0 - 0.5
        bw = (_sigmoid(pw) * 2.0) ** 2 * aw
        bh = (_sigmoid(ph) * 2.0) ** 2 * ah
        b1x1, b1x2 = bx - bw / 2, bx + bw / 2
        b1y1, b1y2 = by - bh / 2, by + bh / 2
        b2x1, b2x2 = tx - tw / 2, tx + tw / 2
        b2y1, b2y2 = ty - th / 2, ty + th / 2
        iw = jnp.clip(jnp.minimum(b1x2, b2x2) - jnp.maximum(b1x1, b2x1), 0.0, None)
        ih = jnp.clip(jnp.minimum(b1y2, b2y2) - jnp.maximum(b1y1, b2y1), 0.0, None)
        inter = iw * ih
        union = jnp.clip(bw * bh + tw * th - inter + eps, eps, None)
        iou = jnp.clip(inter / union, 0.0, 1.0)
        cw = jnp.maximum(b1x2, b2x2) - jnp.minimum(b1x1, b2x1)
        chh = jnp.maximum(b1y2, b2y2) - jnp.minimum(b1y1, b2y1)
        c2 = jnp.clip(cw ** 2 + chh ** 2 + eps, eps, None)
        rho2 = ((b2x1 + b2x2 - b1x1 - b1x2) ** 2
                + (b2y1 + b2y2 - b1y1 - b1y2) ** 2) / c2
        v = (4.0 / math.pi ** 2) * (_atan_pos(tw / (th + eps))
                                    - _atan_pos(bw / (bh + eps))) ** 2
        alpha = v / (v - iou + 1.0 + eps)
        vc = jnp.clip(iou - (rho2 + v * alpha), 0.0, 1.0)
        u = 1.0 - vc
        out_ref[l, 0] = jnp.sum(mask * (u * jnp.sqrt(u)))
        out_ref[l, 1] = jnp.sum(mask)
        out_ref[l, 2] = jnp.sum(mask * xo * vc)
        out_ref[l, 3] = 0.0


def _math(ps0, ps1, ps2, m0, m1, m2):
    return pl.pallas_call(
        _math_body,
        out_shape=jax.ShapeDtypeStruct((3, 4), jnp.float32),
        out_specs=pl.BlockSpec(memory_space=pltpu.SMEM),
    )(ps0, ps1, ps2, m0, m1, m2)


def _softplus(x):
    return jnp.maximum(x, 0.0) + jnp.log1p(jnp.exp(-jnp.abs(x)))


def kernel(p0, p1, p2, targets):
    t6 = targets.T
    g0, g1, g2, m0, m1, m2 = _prep(t6)
    gr = [g.reshape(_NW, _NCHUNK, _CHUNK) for g in (g0, g1, g2)]
    o0, o1, o2 = _sc_gather(p0.reshape(-1), p1.reshape(-1), p2.reshape(-1),
                            *gr)
    obj = _objsum(p0, p1, p2)
    ps = [o.reshape(_NO, _NROW, _NT) for o in (o0, o1, o2)]
    sums = _math(*ps, m0, m1, m2)

    loss0 = jnp.float32(0.0)
    loss1 = jnp.float32(0.0)
    for l, (hh, ww) in enumerate(_HW):
        n = jnp.float32(_B * _NA * hh * ww)
        box = sums[l, 0] / jnp.maximum(sums[l, 1], 1.0)
        objl = (obj[0, l] - sums[l, 2]) / n
        loss0 = loss0 + box * _BAL[l]
        loss1 = loss1 + objl * _BAL[l]
    loss0 = loss0 * 0.05
    loss0 = jnp.where(jnp.isnan(loss0), jnp.float32(0.1), loss0)
    loss1 = jnp.where(jnp.isnan(loss1), jnp.float32(0.1), loss1)
    total = loss0 + loss1
    total = jnp.where(jnp.isnan(total), jnp.float32(1.0), total)
    return (total, lax.stop_gradient(jnp.stack([loss0, loss1])))


# speedup vs baseline: 1.3358x; 1.3358x over previous
import functools
import math

import jax
import jax.numpy as jnp
import numpy as np
from jax import lax
from jax.experimental import pallas as pl
from jax.experimental.pallas import tpu as pltpu
from jax.experimental.pallas import tpu_sc as plsc

_ANCH = np.array(
    [[[16.2, 14.4], [41.1, 33.3], [74.1, 57.6]],
     [[110.4, 84.0], [146.4, 107.1], [180.3, 132.6]],
     [[226.0, 129.3], [214.8, 188.8], [278.2, 173.3]]], dtype=np.float32)
_BAL = (4.0, 1.0, 0.25)
_GAMMA = 1.5
_ANCHOR_T = 4.0
_IMG = 640.0

_B = 32
_NA = 3
_NO = 5
_NT = 512
_HW = ((80, 80), (40, 40), (20, 20))
_SA = [ _ANCH[i] / (_IMG / max(h, w)) for i, (h, w) in enumerate(_HW) ]

_NROW = _NO * _NA
_NCAND = _NROW * _NT
_NW = 32
_GPW = _NO * _NCAND // _NW
_CHUNK = 120
_NCHUNK = _GPW // _CHUNK


def _prep_compute(t6_ref, g0, g1, g2, m0, m1, m2):
    img = t6_ref[0:1, :]
    x = t6_ref[2:3, :]
    y = t6_ref[3:4, :]
    w = t6_ref[4:5, :]
    h = t6_ref[5:6, :]
    r_i = lax.broadcasted_iota(jnp.int32, (_NROW, _NT), 0)
    k = r_i // _NA
    a = r_i % _NA
    b = img.astype(jnp.int32)

    for (gref, mref, (H, W), sa) in ((g0, m0, _HW[0], _SA[0]),
                                     (g1, m1, _HW[1], _SA[1]),
                                     (g2, m2, _HW[2], _SA[2])):
        Wf, Hf = jnp.float32(W), jnp.float32(H)
        gx = x * Wf
        gy = y * Hf
        gw = w * Wf
        gh = h * Hf
        aw = jnp.where(a == 0, sa[0, 0], jnp.where(a == 1, sa[1, 0], sa[2, 0]))
        ah = jnp.where(a == 0, sa[0, 1], jnp.where(a == 1, sa[1, 1], sa[2, 1]))
        rw = gw / aw
        rh = gh / ah
        rr = jnp.maximum(jnp.maximum(rw, 1.0 / rw), jnp.maximum(rh, 1.0 / rh))
        jsel = rr < _ANCHOR_T
        jj = ((gx - jnp.floor(gx)) < 0.5) & (gx > 1.0)
        kk = ((gy - jnp.floor(gy)) < 0.5) & (gy > 1.0)
        gxi = Wf - gx
        gyi = Hf - gy
        ll = ((gxi - jnp.floor(gxi)) < 0.5) & (gxi > 1.0)
        mm = ((gyi - jnp.floor(gyi)) < 0.5) & (gyi > 1.0)
        jm = ((k == 0) | ((k == 1) & jj) | ((k == 2) & kk)
              | ((k == 3) & ll) | ((k == 4) & mm))
        mask = (jm & jsel).astype(jnp.float32)
        offx = jnp.where(k == 1, 0.5, 0.0) + jnp.where(k == 3, -0.5, 0.0)
        offy = jnp.where(k == 2, 0.5, 0.0) + jnp.where(k == 4, -0.5, 0.0)
        gsx = gx - offx
        gsy = gy - offy
        gijx = gsx.astype(jnp.int32)
        gijy = gsy.astype(jnp.int32)
        gi = jnp.clip(gijx, 0, W - 1)
        gj = jnp.clip(gijy, 0, H - 1)
        tx = gx - gijx.astype(jnp.float32)
        ty = gy - gijy.astype(jnp.float32)
        base = ((b * 15 + a * _NO) * H + gj) * W + gi
        ch = lax.broadcasted_iota(jnp.int32, (_NO, _NROW, _NT), 0)
        gref[...] = base[None] + ch * (H * W)
        zero = jnp.zeros((_NROW, _NT), jnp.float32)
        mref[...] = jnp.stack(
            [mask, tx, ty,
             jnp.broadcast_to(gw, (_NROW, _NT)),
             jnp.broadcast_to(gh, (_NROW, _NT)),
             aw, ah, zero], axis=0)


def _prep(t6):
    shp_g = jax.ShapeDtypeStruct((_NO, _NROW, _NT), jnp.int32)
    shp_m = jax.ShapeDtypeStruct((8, _NROW, _NT), jnp.float32)
    return pl.pallas_call(
        _prep_compute,
        out_shape=(shp_g, shp_g, shp_g, shp_m, shp_m, shp_m),
    )(t6)


def _objsum_body(p0r, p1r, p2r, s_ref):
    ci = pl.program_id(0)

    @pl.when(ci == 0)
    def _():
        s_ref[0, 0] = 0.0
        s_ref[0, 1] = 0.0
        s_ref[0, 2] = 0.0
        s_ref[0, 3] = 0.0

    s_ref[0, 0] += jnp.sum(_softplus(p0r[...]))
    s_ref[0, 1] += jnp.sum(_softplus(p1r[...]))
    s_ref[0, 2] += jnp.sum(_softplus(p2r[...]))


def _objsum(p0, p1, p2):
    in_specs = [
        pl.BlockSpec((_B, 1, hh, ww), lambda c: (0, 5 * c + 4, 0, 0))
        for (hh, ww) in _HW
    ]
    return pl.pallas_call(
        _objsum_body,
        grid=(3,),
        in_specs=in_specs,
        out_specs=pl.BlockSpec(memory_space=pltpu.SMEM),
        out_shape=jax.ShapeDtypeStruct((1, 4), jnp.float32),
    )(p0, p1, p2)


def _sc_gather_body(p0h, p1h, p2h, g0h, g1h, g2h, o0h, o1h, o2h,
                    i0, i1, i2, v0, v1, v2, sem):
    wid = lax.axis_index("s") * 2 + lax.axis_index("c")
    trip = ((p0h, g0h, o0h, i0, v0), (p1h, g1h, o1h, i1, v1),
            (p2h, g2h, o2h, i2, v2))
    lds = [pltpu.async_copy(gh.at[wid], iv, sem)
           for (_, gh, _, iv, _) in trip]
    for c in lds:
        c.wait()
    gs = [pltpu.async_copy(ph.at[iv.at[j]], vv.at[j], sem)
          for (ph, _, _, iv, vv) in trip
          for j in range(_NCHUNK)]
    for c in gs:
        c.wait()
    ws = [pltpu.async_copy(vv, oh.at[wid], sem)
          for (_, _, oh, _, vv) in trip]
    for c in ws:
        c.wait()


def _sc_gather(p0f, p1f, p2f, g0, g1, g2):
    mesh = plsc.VectorSubcoreMesh(core_axis_name="c", subcore_axis_name="s")
    out = jax.ShapeDtypeStruct((_NW, _NCHUNK, _CHUNK), jnp.float32)
    fn = functools.partial(
        pl.kernel,
        out_type=[out, out, out],
        mesh=mesh,
        scratch_types=[
            pltpu.VMEM((_NCHUNK, _CHUNK), jnp.int32),
            pltpu.VMEM((_NCHUNK, _CHUNK), jnp.int32),
            pltpu.VMEM((_NCHUNK, _CHUNK), jnp.int32),
            pltpu.VMEM((_NCHUNK, _CHUNK), jnp.float32),
            pltpu.VMEM((_NCHUNK, _CHUNK), jnp.float32),
            pltpu.VMEM((_NCHUNK, _CHUNK), jnp.float32),
            pltpu.SemaphoreType.DMA,
        ],
    )(_sc_gather_body)
    return fn(p0f, p1f, p2f, g0, g1, g2)


def _sigmoid(v):
    return 1.0 / (1.0 + jnp.exp(-v))


_ATAN_C = (1.0, -0.3333312, 0.19993716, -0.14213197, 0.10681421,
           -0.0759681, 0.0438556, -0.01682745, 0.003049968)


def _atan_pos(x):
    inv = x > 1.0
    z = jnp.where(inv, 1.0 / x, x)
    z2 = z * z
    acc = jnp.full_like(z2, _ATAN_C[-1])
    for c in _ATAN_C[-2::-1]:
        acc = acc * z2 + c
    r = z * acc
    return jnp.where(inv, (math.pi / 2) - r, r)


def _math_body(ps0, ps1, ps2, m0, m1, m2, out_ref):
    eps = jnp.float32(1e-7)
    for l, (psr, mr) in enumerate(((ps0, m0), (ps1, m1), (ps2, m2))):
        px, py, pw, ph, xo = (psr[c] for c in range(_NO))
        mask, tx, ty, tw, th, aw, ah = (mr[c] for c in range(7))
        bx = _sigmoid(px) * 2.0 - 0.5
        by = _sigmoid(py) * 2.0 - 0.5
        bw = (_sigmoid(pw) * 2.0) ** 2 * aw
        bh = (_sigmoid(ph) * 2.0) ** 2 * ah
        b1x1, b1x2 = bx - bw / 2, bx + bw / 2
        b1y1, b1y2 = by - bh / 2, by + bh / 2
        b2x1, b2x2 = tx - tw / 2, tx + tw / 2
        b2y1, b2y2 = ty - th / 2, ty + th / 2
        iw = jnp.clip(jnp.minimum(b1x2, b2x2) - jnp.maximum(b1x1, b2x1), 0.0, None)
        ih = jnp.clip(jnp.minimum(b1y2, b2y2) - jnp.maximum(b1y1, b2y1), 0.0, None)
        inter = iw * ih
        union = jnp.clip(bw * bh + tw * th - inter + eps, eps, None)
        iou = jnp.clip(inter / union, 0.0, 1.0)
        cw = jnp.maximum(b1x2, b2x2) - jnp.minimum(b1x1, b2x1)
        chh = jnp.maximum(b1y2, b2y2) - jnp.minimum(b1y1, b2y1)
        c2 = jnp.clip(cw ** 2 + chh ** 2 + eps, eps, None)
        rho2 = ((b2x1 + b2x2 - b1x1 - b1x2) ** 2
                + (b2y1 + b2y2 - b1y1 - b1y2) ** 2) / c2
        v = (4.0 / math.pi ** 2) * (_atan_pos(tw / (th + eps))
                                    - _atan_pos(bw / (bh + eps))) ** 2
        alpha = v / (v - iou + 1.0 + eps)
        vc = jnp.clip(iou - (rho2 + v * alpha), 0.0, 1.0)
        u = 1.0 - vc
        out_ref[l, 0] = jnp.sum(mask * (u * jnp.sqrt(u)))
        out_ref[l, 1] = jnp.sum(mask)
        out_ref[l, 2] = jnp.sum(mask * xo * vc)
        out_ref[l, 3] = 0.0


def _math(ps0, ps1, ps2, m0, m1, m2):
    return pl.pallas_call(
        _math_body,
        out_shape=jax.ShapeDtypeStruct((3, 4), jnp.float32),
        out_specs=pl.BlockSpec(memory_space=pltpu.SMEM),
    )(ps0, ps1, ps2, m0, m1, m2)


def _softplus(x):
    return jnp.maximum(x, 0.0) + jnp.log1p(jnp.exp(-jnp.abs(x)))


def kernel(p0, p1, p2, targets):
    t6 = targets.T
    g0, g1, g2, m0, m1, m2 = _prep(t6)
    gr = [g.reshape(_NW, _NCHUNK, _CHUNK) for g in (g0, g1, g2)]
    o0, o1, o2 = _sc_gather(p0.reshape(-1), p1.reshape(-1), p2.reshape(-1),
                            *gr)
    obj = _objsum(p0, p1, p2)
    ps = [o.reshape(_NO, _NROW, _NT) for o in (o0, o1, o2)]
    sums = _math(*ps, m0, m1, m2)

    loss0 = jnp.float32(0.0)
    loss1 = jnp.float32(0.0)
    for l, (hh, ww) in enumerate(_HW):
        n = jnp.float32(_B * _NA * hh * ww)
        box = sums[l, 0] / jnp.maximum(sums[l, 1], 1.0)
        objl = (obj[0, l] - sums[l, 2]) / n
        loss0 = loss0 + box * _BAL[l]
        loss1 = loss1 + objl * _BAL[l]
    loss0 = loss0 * 0.05
    loss0 = jnp.where(jnp.isnan(loss0), jnp.float32(0.1), loss0)
    loss1 = jnp.where(jnp.isnan(loss1), jnp.float32(0.1), loss1)
    total = loss0 + loss1
    total = jnp.where(jnp.isnan(total), jnp.float32(1.0), total)
    return (total, lax.stop_gradient(jnp.stack([loss0, loss1])))


# speedup vs baseline: 1.4825x; 1.1098x over previous
import functools
import math

import jax
import jax.numpy as jnp
import numpy as np
from jax import lax
from jax.experimental import pallas as pl
from jax.experimental.pallas import tpu as pltpu
from jax.experimental.pallas import tpu_sc as plsc

_ANCH = np.array(
    [[[16.2, 14.4], [41.1, 33.3], [74.1, 57.6]],
     [[110.4, 84.0], [146.4, 107.1], [180.3, 132.6]],
     [[226.0, 129.3], [214.8, 188.8], [278.2, 173.3]]], dtype=np.float32)
_BAL = (4.0, 1.0, 0.25)
_GAMMA = 1.5
_ANCHOR_T = 4.0
_IMG = 640.0

_B = 32
_NA = 3
_NO = 5
_NT = 512
_HW = ((80, 80), (40, 40), (20, 20))
_SA = [ _ANCH[i] / (_IMG / max(h, w)) for i, (h, w) in enumerate(_HW) ]

_NROW = _NO * _NA
_NCAND = _NROW * _NT
_NPAD = 64 * 128
_NW = 32
_GPW = _NO * _NPAD // _NW
_CHUNK = 128
_NCHUNK = _GPW // _CHUNK


def _prep_compute(t_ref, g0, g1, g2, m0, m1, m2):
    def field(f):
        blk = t_ref[pl.ds(8 * f, 8), :]
        return jnp.concatenate([blk[0:4]] * 16, axis=0)

    img = field(0)
    x = field(2)
    y = field(3)
    w = field(4)
    h = field(5)
    r_i = lax.broadcasted_iota(jnp.int32, (64, 128), 0)
    kr = r_i // 4
    k = kr // _NA
    a = kr - _NA * k
    b = img.astype(jnp.int32)

    for (gref, mref, (H, W), sa) in ((g0, m0, _HW[0], _SA[0]),
                                     (g1, m1, _HW[1], _SA[1]),
                                     (g2, m2, _HW[2], _SA[2])):
        Wf, Hf = jnp.float32(W), jnp.float32(H)
        gx = x * Wf
        gy = y * Hf
        gw = w * Wf
        gh = h * Hf
        aw = jnp.where(a == 0, sa[0, 0], jnp.where(a == 1, sa[1, 0], sa[2, 0]))
        ah = jnp.where(a == 0, sa[0, 1], jnp.where(a == 1, sa[1, 1], sa[2, 1]))
        rw = gw / aw
        rh = gh / ah
        rr = jnp.maximum(jnp.maximum(rw, 1.0 / rw), jnp.maximum(rh, 1.0 / rh))
        jsel = rr < _ANCHOR_T
        jj = ((gx - jnp.floor(gx)) < 0.5) & (gx > 1.0)
        kk = ((gy - jnp.floor(gy)) < 0.5) & (gy > 1.0)
        gxi = Wf - gx
        gyi = Hf - gy
        ll = ((gxi - jnp.floor(gxi)) < 0.5) & (gxi > 1.0)
        mm = ((gyi - jnp.floor(gyi)) < 0.5) & (gyi > 1.0)
        jm = ((k == 0) | ((k == 1) & jj) | ((k == 2) & kk)
              | ((k == 3) & ll) | ((k == 4) & mm))
        mask = (jm & jsel).astype(jnp.float32)
        offx = jnp.where(k == 1, 0.5, 0.0) + jnp.where(k == 3, -0.5, 0.0)
        offy = jnp.where(k == 2, 0.5, 0.0) + jnp.where(k == 4, -0.5, 0.0)
        gsx = gx - offx
        gsy = gy - offy
        gijx = gsx.astype(jnp.int32)
        gijy = gsy.astype(jnp.int32)
        gi = jnp.clip(gijx, 0, W - 1)
        gj = jnp.clip(gijy, 0, H - 1)
        tx = gx - gijx.astype(jnp.float32)
        ty = gy - gijy.astype(jnp.float32)
        base = ((b * 15 + a * _NO) * H + gj) * W + gi
        gref[...] = jnp.concatenate(
            [base + c * (H * W) for c in range(_NO)], axis=0)
        mref[...] = jnp.concatenate(
            [mask, tx, ty, gw, gh, aw, ah,
             jnp.zeros((64, 128), jnp.float32)], axis=0)


def _prep(t48):
    shp_g = jax.ShapeDtypeStruct((_NO * 64, 128), jnp.int32)
    shp_m = jax.ShapeDtypeStruct((8 * 64, 128), jnp.float32)
    return pl.pallas_call(
        _prep_compute,
        out_shape=(shp_g, shp_g, shp_g, shp_m, shp_m, shp_m),
    )(t48)


def _objsum_body(p0r, p1r, p2r, s_ref):
    ci = pl.program_id(0)

    @pl.when(ci == 0)
    def _():
        s_ref[0, 0] = 0.0
        s_ref[0, 1] = 0.0
        s_ref[0, 2] = 0.0
        s_ref[0, 3] = 0.0

    s_ref[0, 0] += jnp.sum(_softplus(p0r[...]))
    s_ref[0, 1] += jnp.sum(_softplus(p1r[...]))
    s_ref[0, 2] += jnp.sum(_softplus(p2r[...]))


def _objsum(p0, p1, p2):
    in_specs = [
        pl.BlockSpec((_B, 1, hh, ww), lambda c: (0, 5 * c + 4, 0, 0))
        for (hh, ww) in _HW
    ]
    return pl.pallas_call(
        _objsum_body,
        grid=(3,),
        in_specs=in_specs,
        out_specs=pl.BlockSpec(memory_space=pltpu.SMEM),
        out_shape=jax.ShapeDtypeStruct((1, 4), jnp.float32),
    )(p0, p1, p2)


def _sc_gather_body(p0h, p1h, p2h, g0h, g1h, g2h, o0h, o1h, o2h,
                    i0, i1, i2, v0, v1, v2, sem):
    wid = lax.axis_index("s") * 2 + lax.axis_index("c")
    base = wid * _GPW
    trip = ((p0h, g0h, o0h, i0, v0), (p1h, g1h, o1h, i1, v1),
            (p2h, g2h, o2h, i2, v2))
    lds = [pltpu.async_copy(gh.at[pl.ds(base, _GPW)], iv, sem)
           for (_, gh, _, iv, _) in trip]
    for c in lds:
        c.wait()
    gs = [pltpu.async_copy(ph.at[iv.at[pl.ds(j * _CHUNK, _CHUNK)]],
                           vv.at[pl.ds(j * _CHUNK, _CHUNK)], sem)
          for (ph, _, _, iv, vv) in trip
          for j in range(_NCHUNK)]
    for c in gs:
        c.wait()
    ws = [pltpu.async_copy(vv, oh.at[pl.ds(base, _GPW)], sem)
          for (_, _, oh, _, vv) in trip]
    for c in ws:
        c.wait()


def _sc_gather(p0f, p1f, p2f, g0, g1, g2):
    mesh = plsc.VectorSubcoreMesh(core_axis_name="c", subcore_axis_name="s")
    out = jax.ShapeDtypeStruct((_NO * _NPAD,), jnp.float32)
    fn = functools.partial(
        pl.kernel,
        out_type=[out, out, out],
        mesh=mesh,
        scratch_types=[
            pltpu.VMEM((_GPW,), jnp.int32),
            pltpu.VMEM((_GPW,), jnp.int32),
            pltpu.VMEM((_GPW,), jnp.int32),
            pltpu.VMEM((_GPW,), jnp.float32),
            pltpu.VMEM((_GPW,), jnp.float32),
            pltpu.VMEM((_GPW,), jnp.float32),
            pltpu.SemaphoreType.DMA,
        ],
    )(_sc_gather_body)
    return fn(p0f, p1f, p2f, g0.reshape(-1), g1.reshape(-1), g2.reshape(-1))


def _sigmoid(v):
    return 1.0 / (1.0 + jnp.exp(-v))


_ATAN_C = (1.0, -0.3333312, 0.19993716, -0.14213197, 0.10681421,
           -0.0759681, 0.0438556, -0.01682745, 0.003049968)


def _atan_pos(x):
    inv = x > 1.0
    z = jnp.where(inv, 1.0 / x, x)
    z2 = z * z
    acc = jnp.full_like(z2, _ATAN_C[-1])
    for c in _ATAN_C[-2::-1]:
        acc = acc * z2 + c
    r = z * acc
    return jnp.where(inv, (math.pi / 2) - r, r)


def _math_body(ps0, ps1, ps2, m0, m1, m2, out_ref):
    eps = jnp.float32(1e-7)
    for l, (psr, mr) in enumerate(((ps0, m0), (ps1, m1), (ps2, m2))):
        px, py, pw, ph, xo = (psr[pl.ds(c * 64, 64), :] for c in range(_NO))
        mask, tx, ty, tw, th, aw, ah = (mr[pl.ds(c * 64, 64), :]
                                        for c in range(7))
        bx = _sigmoid(px) * 2.0 - 0.5
        by = _sigmoid(py) * 2.0 - 0.5
        bw = (_sigmoid(pw) * 2.0) ** 2 * aw
        bh = (_sigmoid(ph) * 2.0) ** 2 * ah
        b1x1, b1x2 = bx - bw / 2, bx + bw / 2
        b1y1, b1y2 = by - bh / 2, by + bh / 2
        b2x1, b2x2 = tx - tw / 2, tx + tw / 2
        b2y1, b2y2 = ty - th / 2, ty + th / 2
        iw = jnp.clip(jnp.minimum(b1x2, b2x2) - jnp.maximum(b1x1, b2x1), 0.0, None)
        ih = jnp.clip(jnp.minimum(b1y2, b2y2) - jnp.maximum(b1y1, b2y1), 0.0, None)
        inter = iw * ih
        union = jnp.clip(bw * bh + tw * th - inter + eps, eps, None)
        iou = jnp.clip(inter / union, 0.0, 1.0)
        cw = jnp.maximum(b1x2, b2x2) - jnp.minimum(b1x1, b2x1)
        chh = jnp.maximum(b1y2, b2y2) - jnp.minimum(b1y1, b2y1)
        c2 = jnp.clip(cw ** 2 + chh ** 2 + eps, eps, None)
        rho2 = ((b2x1 + b2x2 - b1x1 - b1x2) ** 2
                + (b2y1 + b2y2 - b1y1 - b1y2) ** 2) / c2
        v = (4.0 / math.pi ** 2) * (_atan_pos(tw / (th + eps))
                                    - _atan_pos(bw / (bh + eps))) ** 2
        alpha = v / (v - iou + 1.0 + eps)
        vc = jnp.clip(iou - (rho2 + v * alpha), 0.0, 1.0)
        u = 1.0 - vc
        out_ref[l, 0] = jnp.sum(mask * (u * jnp.sqrt(u)))
        out_ref[l, 1] = jnp.sum(mask)
        out_ref[l, 2] = jnp.sum(mask * xo * vc)
        out_ref[l, 3] = 0.0


def _math(ps0, ps1, ps2, m0, m1, m2):
    return pl.pallas_call(
        _math_body,
        out_shape=jax.ShapeDtypeStruct((3, 4), jnp.float32),
        out_specs=pl.BlockSpec(memory_space=pltpu.SMEM),
    )(ps0, ps1, ps2, m0, m1, m2)


def _softplus(x):
    return jnp.maximum(x, 0.0) + jnp.log1p(jnp.exp(-jnp.abs(x)))


def kernel(p0, p1, p2, targets):
    t48 = jnp.pad(targets.T.reshape(6, 4, 128),
                  ((0, 0), (0, 4), (0, 0))).reshape(48, 128)
    g0, g1, g2, m0, m1, m2 = _prep(t48)
    o0, o1, o2 = _sc_gather(p0.reshape(-1), p1.reshape(-1), p2.reshape(-1),
                            g0, g1, g2)
    obj = _objsum(p0, p1, p2)
    ps = [o.reshape(_NO * 64, 128) for o in (o0, o1, o2)]
    sums = _math(*ps, m0, m1, m2)

    loss0 = jnp.float32(0.0)
    loss1 = jnp.float32(0.0)
    for l, (hh, ww) in enumerate(_HW):
        n = jnp.float32(_B * _NA * hh * ww)
        box = sums[l, 0] / jnp.maximum(sums[l, 1], 1.0)
        objl = (obj[0, l] - sums[l, 2]) / n
        loss0 = loss0 + box * _BAL[l]
        loss1 = loss1 + objl * _BAL[l]
    loss0 = loss0 * 0.05
    loss0 = jnp.where(jnp.isnan(loss0), jnp.float32(0.1), loss0)
    loss1 = jnp.where(jnp.isnan(loss1), jnp.float32(0.1), loss1)
    total = loss0 + loss1
    total = jnp.where(jnp.isnan(total), jnp.float32(1.0), total)
    return (total, lax.stop_gradient(jnp.stack([loss0, loss1])))
